# baseline (speedup 1.0000x reference)
import functools

import jax
import jax.numpy as jnp
from jax import lax
from jax.experimental import pallas as pl
from jax.experimental.pallas import tpu as pltpu
from jax.experimental.pallas import tpu_sc as plsc

N = 10000
NP = 10240
E = 320000
NC, NS = 2, 16
NT = NC * NS
EPT = E // NT
SUB = 100
B = 200
NSUB = B // SUB
NBLK = EPT // B
RPT = NP // NS
ZR = 64
BN = 1024
GRID = NP // BN

f32 = jnp.float32
i32 = jnp.int32

_MESH = plsc.VectorSubcoreMesh(core_axis_name="c", subcore_axis_name="s")


def _b16(v):
    return jnp.full((16,), v, i32)


def _worker(cid, sid):
    return sid * NC + cid


def _make_zkernel(SW):
    dup = SW == 16

    @functools.partial(
        pl.kernel,
        out_type=(
            jax.ShapeDtypeStruct((E, SW), f32),
            jax.ShapeDtypeStruct((NC, NP, SW), f32),
        ),
        mesh=_MESH,
        compiler_params=pltpu.CompilerParams(needs_layout_passes=False, use_tc_tiling_on_sc=False),
        scratch_types=[
            pltpu.VMEM((NSUB, SUB), i32),
            pltpu.VMEM((NSUB, SUB), i32),
            pltpu.VMEM((B,), f32),
            pltpu.VMEM((B, 16), f32),
            pltpu.VMEM((B, SW), f32),
            pltpu.VMEM((16,), f32),
            pltpu.VMEM((ZR, SW), f32),
            pltpu.VMEM_SHARED((NP, SW), f32),
            pltpu.SemaphoreType.DMA,
        ],
    )
    def zkernel(a_hbm, et_hbm, srcr, dstr, awe_hbm, z_hbm, s_hbm,
                src_v, dst_v, et_v, a_rows, srows, awe_v, zero_v, s_slab,
                sem):
        cid = lax.axis_index("c")
        sid = lax.axis_index("s")
        wid = _worker(cid, sid)
        pltpu.sync_copy(awe_hbm, awe_v)

        @pl.loop(0, ZR)
        def _zero(r):
            for c in range(SW // 16):
                zero_v[r, pl.ds(c * 16, 16)] = jnp.zeros((16,), f32)

        for i in range(RPT // ZR):
            pltpu.sync_copy(zero_v, s_slab.at[pl.ds(sid * RPT + i * ZR, ZR)])
        plsc.subcore_barrier()

        lane = lax.iota(i32, 16)
        awe = awe_v[...]

        @pl.loop(0, NBLK)
        def _blk(b):
            e0 = wid * EPT + b * B
            pltpu.sync_copy(srcr.at[wid, b], src_v)
            pltpu.sync_copy(dstr.at[wid, b], dst_v)
            pltpu.sync_copy(et_hbm.at[pl.ds(e0, B)], et_v)
            cps = [
                pltpu.async_copy(a_hbm.at[src_v.at[j]],
                                 a_rows.at[pl.ds(j * SUB, SUB)], sem)
                for j in range(NSUB)
            ]
            for cp in cps:
                cp.wait()

            @plsc.parallel_loop(0, B, unroll=8)
            def _edge(e):
                a16 = a_rows[e, :]
                etb = plsc.load_gather(et_v, [_b16(e)])
                lg = a16 + etb * awe
                lg = jnp.where(lg < 0.0, lg * 0.2, lg)
                z16 = jnp.exp(lg)
                if dup:
                    srows[e, :] = jnp.where(lane < 8, z16, z16 * etb)
                else:
                    srows[e, pl.ds(0, 16)] = z16
                    srows[e, pl.ds(16, 16)] = z16 * etb

            for j in range(NSUB):
                pltpu.sync_copy(srows.at[pl.ds(j * SUB, SUB)],
                                s_slab.at[dst_v.at[j]], add=True)
            pltpu.sync_copy(srows, z_hbm.at[pl.ds(e0, B)])

        plsc.subcore_barrier()
        pltpu.sync_copy(s_slab.at[pl.ds(sid * RPT, RPT)],
                        s_hbm.at[cid, pl.ds(sid * RPT, RPT)])

    return zkernel


def _make_scale_kernel(K, hpc, SW):
    W = 128
    BS = 100
    NB = EPT // BS

    @functools.partial(
        pl.kernel,
        out_type=jax.ShapeDtypeStruct((NC, K, NP, W), f32),
        mesh=_MESH,
        compiler_params=pltpu.CompilerParams(needs_layout_passes=False, use_tc_tiling_on_sc=False),
        scratch_types=[
            pltpu.VMEM((2, BS), i32),
            pltpu.VMEM((2, BS), i32),
            pltpu.VMEM((2, BS, SW), f32),
            pltpu.VMEM((2, BS, W), f32),
            pltpu.VMEM((ZR, W), f32),
            pltpu.VMEM_SHARED((NP, W), f32),
            pltpu.SemaphoreType.DMA,
            pltpu.SemaphoreType.DMA,
        ],
    )
    def skernel(y_hbm, z_hbm, srcr, dstr, u_hbm,
                src_v, dst_v, z_v, rows_v, zero_v, u_slab, gsem, ssem):
        cid = lax.axis_index("c")
        sid = lax.axis_index("s")
        wid = _worker(cid, sid)

        @pl.loop(0, ZR)
        def _zero(r):
            for c in range(W // 16):
                zero_v[r, pl.ds(c * 16, 16)] = jnp.zeros((16,), f32)

        def load_idx(g, p):
            pltpu.sync_copy(srcr.at[wid, g, p], src_v.at[p])
            pltpu.sync_copy(dstr.at[wid, g, p], dst_v.at[p])
            e0 = wid * EPT + (g * NSUB + p) * BS
            pltpu.sync_copy(z_hbm.at[pl.ds(e0, BS)], z_v.at[p])

        def scale(k, p):
            @plsc.parallel_loop(0, BS, step=10)
            def _grp(e0):
                for jj in range(10):
                    e = e0 + jj
                    if hpc == 1:
                        zb = plsc.load_gather(
                            z_v.at[p], [_b16(e), _b16(k)])
                        for c in range(8):
                            rows_v[p, e, pl.ds(c * 16, 16)] = (
                                rows_v[p, e, pl.ds(c * 16, 16)] * zb)
                    else:
                        zb0 = plsc.load_gather(
                            z_v.at[p], [_b16(e), _b16(2 * k)])
                        zb1 = plsc.load_gather(
                            z_v.at[p], [_b16(e), _b16(2 * k + 1)])
                        for c in range(4):
                            rows_v[p, e, pl.ds(c * 16, 16)] = (
                                rows_v[p, e, pl.ds(c * 16, 16)] * zb0)
                        for c in range(4, 8):
                            rows_v[p, e, pl.ds(c * 16, 16)] = (
                                rows_v[p, e, pl.ds(c * 16, 16)] * zb1)

        for k in range(K):
            for i in range(RPT // ZR):
                pltpu.sync_copy(
                    zero_v, u_slab.at[pl.ds(sid * RPT + i * ZR, ZR)])
            plsc.subcore_barrier()

            load_idx(0, 0)
            pltpu.async_copy(y_hbm.at[k].at[src_v.at[0]],
                             rows_v.at[0], gsem)

            @pl.loop(0, NB // 2)
            def _pair(g):
                for p in range(2):
                    b = g * 2 + p
                    q = 1 - p
                    pltpu.make_async_copy(y_hbm.at[k].at[src_v.at[p]],
                                          rows_v.at[p], gsem).wait()

                    @pl.when(b >= 1)
                    def _wait_prev_scatter():
                        pltpu.make_async_copy(
                            rows_v.at[q],
                            u_slab.at[dst_v.at[q]], ssem).wait()

                    @pl.when(b + 1 < NB)
                    def _prefetch():
                        gq = (g * 2 + p + 1) // 2
                        pq = (g * 2 + p + 1) % 2
                        load_idx(gq, pq)
                        pltpu.async_copy(y_hbm.at[k].at[src_v.at[q]],
                                         rows_v.at[q], gsem)

                    scale(k, p)
                    pltpu.async_copy(rows_v.at[p],
                                     u_slab.at[dst_v.at[p]], ssem,
                                     add=True)

            pltpu.make_async_copy(rows_v.at[1],
                                  u_slab.at[dst_v.at[1]], ssem).wait()
            plsc.subcore_barrier()
            pltpu.sync_copy(u_slab.at[pl.ds(sid * RPT, RPT)],
                            u_hbm.at[cid, k, pl.ds(sid * RPT, RPT)])
            plsc.subcore_barrier()

    return skernel


@functools.partial(
    pl.kernel,
    out_type=(
        jax.ShapeDtypeStruct((NC, NP, 32), f32),
        jax.ShapeDtypeStruct((NC, NP, 16), f32),
    ),
    mesh=_MESH,
    compiler_params=pltpu.CompilerParams(needs_layout_passes=False, use_tc_tiling_on_sc=False),
    scratch_types=[
        pltpu.VMEM((NSUB, SUB), i32),
        pltpu.VMEM((NSUB, SUB), i32),
        pltpu.VMEM((B, 16), f32),
        pltpu.VMEM((B, 32), f32),
        pltpu.VMEM((B, 16), f32),
        pltpu.VMEM((ZR, 32), f32),
        pltpu.VMEM_SHARED((NP, 32), f32),
        pltpu.VMEM_SHARED((NP, 16), f32),
        pltpu.SemaphoreType.DMA,
    ],
)
def _l3_kernel(a_hbm, y_hbm, srcr, dstr, u_hbm, s_hbm,
               src_v, dst_v, a_rows, rows_v, srows, zero_v, u_slab, s_slab,
               sem):
    cid = lax.axis_index("c")
    sid = lax.axis_index("s")
    wid = _worker(cid, sid)

    @pl.loop(0, ZR)
    def _zero(r):
        for c in range(2):
            zero_v[r, pl.ds(c * 16, 16)] = jnp.zeros((16,), f32)

    for i in range(RPT // ZR):
        pltpu.sync_copy(zero_v, u_slab.at[pl.ds(sid * RPT + i * ZR, ZR)])
        pltpu.sync_copy(zero_v.at[:, pl.ds(0, 16)],
                        s_slab.at[pl.ds(sid * RPT + i * ZR, ZR)])
    plsc.subcore_barrier()

    lane = lax.iota(i32, 16)

    @pl.loop(0, NBLK)
    def _blk(b):
        pltpu.sync_copy(srcr.at[wid, b], src_v)
        pltpu.sync_copy(dstr.at[wid, b], dst_v)
        cps = [
            pltpu.async_copy(a_hbm.at[src_v.at[j]],
                             a_rows.at[pl.ds(j * SUB, SUB)], sem)
            for j in range(NSUB)
        ] + [
            pltpu.async_copy(y_hbm.at[src_v.at[j]],
                             rows_v.at[pl.ds(j * SUB, SUB)], sem)
            for j in range(NSUB)
        ]
        for cp in cps:
            cp.wait()

        @plsc.parallel_loop(0, B, unroll=8)
        def _edge(e):
            a16 = a_rows[e, :]
            lg = jnp.where(a16 < 0.0, a16 * 0.2, a16)
            z16 = jnp.exp(lg)
            srows[e, :] = jnp.where(lane < 1, z16, jnp.zeros((16,), f32))
            rows_v[e, pl.ds(0, 16)] = rows_v[e, pl.ds(0, 16)] * z16
            rows_v[e, pl.ds(16, 16)] = rows_v[e, pl.ds(16, 16)] * z16

        for j in range(NSUB):
            pltpu.sync_copy(rows_v.at[pl.ds(j * SUB, SUB)],
                            u_slab.at[dst_v.at[j]], add=True)
            pltpu.sync_copy(srows.at[pl.ds(j * SUB, SUB)],
                            s_slab.at[dst_v.at[j]], add=True)

    plsc.subcore_barrier()
    pltpu.sync_copy(u_slab.at[pl.ds(sid * RPT, RPT)],
                    u_hbm.at[cid, pl.ds(sid * RPT, RPT)])
    pltpu.sync_copy(s_slab.at[pl.ds(sid * RPT, RPT)],
                    s_hbm.at[cid, pl.ds(sid * RPT, RPT)])


def _elu(x):
    return jnp.where(x > 0.0, x, jnp.exp(x) - 1.0)


def _combine(u, s, We, heads, C, eps=1e-16):
    s0 = s[:, :heads]
    acc = jnp.zeros((u.shape[1], C), f32)
    for h in range(heads):
        if C == 128:
            uh = u[h]
        else:
            k, half = h // 2, h % 2
            uh = u[k][:, half * 64:half * 64 + 64]
        if We is not None:
            s1 = s[:, heads:]
            uh = uh + s1[:, h:h + 1] * We[0, h * C:(h + 1) * C][None, :]
        acc = acc + uh / (s0[:, h:h + 1] + eps)
    return acc / heads


def _tc0_body(nodes_ref, emb_ref, Wm0_ref, b0_ref, att0_ref,
              y0_ref, a0_ref, x0_ref):
    nodes = nodes_ref[0, 0, :]
    oh = (nodes[:, None] == lax.broadcasted_iota(i32, (BN, 50), 1))
    xb = jnp.dot(oh.astype(f32), emb_ref[...],
                 preferred_element_type=f32)
    y = jnp.dot(xb, Wm0_ref[...], preferred_element_type=f32) + b0_ref[0]
    for k in range(8):
        y0_ref[k] = y[:, k * 128:(k + 1) * 128]
    an = jnp.sum(y.reshape(BN, 8, 128) * att0_ref[...][None], axis=-1)
    a0_ref[...] = jnp.concatenate([an, an], axis=1)
    x0_ref[...] = xb


def _tc1_body(u_ref, s_ref, x0_ref, We0_ref, Wm1_ref, b1_ref, att1_ref,
              Ws1_ref, bs1_ref, y1_ref, a1_ref, xs1_ref):
    u = u_ref[0] + u_ref[1]
    s = s_ref[0] + s_ref[1]
    out = _combine(u, s, We0_ref[...], 8, 128)
    x1 = _elu(out + x0_ref[...])
    y = jnp.dot(x1, Wm1_ref[...], preferred_element_type=f32) + b1_ref[0]
    for k in range(4):
        y1_ref[k] = y[:, k * 128:(k + 1) * 128]
    an = jnp.sum(y.reshape(BN, 8, 64) * att1_ref[...][None], axis=-1)
    a1_ref[...] = jnp.concatenate([an, an], axis=1)
    xs1_ref[...] = jnp.dot(x1, Ws1_ref[...],
                           preferred_element_type=f32) + bs1_ref[0]


def _tc2_body(u_ref, s_ref, xs1_ref, We1_ref, Wm2_ref, b2_ref, att2_ref,
              y2_ref, a2_ref, x2_ref):
    u = u_ref[0] + u_ref[1]
    s = s_ref[0] + s_ref[1]
    out = _combine(u, s, We1_ref[...], 8, 64)
    x2 = _elu(out + xs1_ref[...])
    y = jnp.dot(x2, Wm2_ref[...], preferred_element_type=f32) + b2_ref[0]
    for k in range(8):
        y2_ref[k] = y[:, k * 128:(k + 1) * 128]
    a2_ref[...] = jnp.sum(y.reshape(BN, 16, 64) * att2_ref[...][None],
                          axis=-1)
    x2_ref[...] = x2


def _tc3_body(u_ref, s_ref, x2_ref, We2_ref, Wm3_ref, bm3_ref, att3_ref,
              Ws3_ref, bs3_ref, y3_ref, a3_ref, xs3_ref):
    u = u_ref[0] + u_ref[1]
    s = s_ref[0] + s_ref[1]
    out = _combine(u, s, We2_ref[...], 16, 64)
    x3 = _elu(out + x2_ref[...])
    y = jnp.dot(x3, Wm3_ref[...], preferred_element_type=f32) + bm3_ref[0]
    y3_ref[...] = y
    an = jnp.sum(y.reshape(BN, 1, 32) * att3_ref[...][None], axis=-1)
    a3_ref[...] = jnp.broadcast_to(an, (BN, 16))
    xs3_ref[...] = jnp.dot(x3, Ws3_ref[...],
                           preferred_element_type=f32) + bs3_ref[0]


def _tc4_body(u_ref, s_ref, xs3_ref, Wc_ref, bc_ref, crit_ref, h_ref):
    u = u_ref[0] + u_ref[1]
    s0 = s_ref[0, :, 0:1] + s_ref[1, :, 0:1]
    h = u / (s0 + 1e-16) + xs3_ref[...]
    h_ref[...] = h
    crit_ref[...] = jnp.dot(h, Wc_ref[...],
                            preferred_element_type=f32) + bc_ref[0]


def _row_spec(*shape):
    nd = len(shape)

    def im(i):
        return tuple(0 for _ in range(nd - 2)) + (i, 0)

    return pl.BlockSpec(shape, im)


def _full_spec(*shape):
    nd = len(shape)

    def im(i):
        return tuple(0 for _ in range(nd))

    return pl.BlockSpec(shape, im)


_zk16 = _make_zkernel(16)
_zk32 = _make_zkernel(32)
_sk_l0 = _make_scale_kernel(8, 1, 16)
_sk_l1 = _make_scale_kernel(4, 2, 16)
_sk_l2 = _make_scale_kernel(8, 2, 32)


def kernel(nodes, edges, edge_type, emb, Wm0, bm0, We0, be0, att0, Wm1, bm1,
           We1, be1, att1, Ws1, bs1, Wm2, bm2, We2, be2, att2, Wm3, bm3,
           att3, Ws3, bs3, Wc, bc):
    src_r = edges[0].astype(i32).reshape(NT, NBLK, NSUB, SUB)
    dst_r = edges[1].astype(i32).reshape(NT, NBLK, NSUB, SUB)
    et = edge_type.astype(f32)

    b0 = (bm0 + be0).reshape(1, 1024)
    b1 = (bm1 + be1).reshape(1, 512)
    b2 = (bm2 + be2).reshape(1, 1024)
    awe0 = jnp.sum(We0.reshape(8, 128) * att0, axis=-1)
    awe0 = jnp.concatenate([awe0, awe0])
    awe1 = jnp.sum(We1.reshape(8, 64) * att1, axis=-1)
    awe1 = jnp.concatenate([awe1, awe1])
    awe2 = jnp.sum(We2.reshape(16, 64) * att2, axis=-1)
    nodes3 = jnp.pad(nodes.astype(i32), (0, NP - N)).reshape(GRID, 1, BN)

    y0, a0, x0 = pl.pallas_call(
        _tc0_body,
        grid=(GRID,),
        in_specs=[
            pl.BlockSpec((1, 1, BN), lambda i: (i, 0, 0)),
            _full_spec(50, 128),
            _full_spec(128, 1024),
            _full_spec(1, 1024),
            _full_spec(8, 128),
        ],
        out_specs=[
            _row_spec(8, BN, 128),
            _row_spec(BN, 16),
            _row_spec(BN, 128),
        ],
        out_shape=[
            jax.ShapeDtypeStruct((8, NP, 128), f32),
            jax.ShapeDtypeStruct((NP, 16), f32),
            jax.ShapeDtypeStruct((NP, 128), f32),
        ],
    )(nodes3, emb, Wm0, b0, att0)

    z0, s0p = _zk16(a0, et, src_r, dst_r, awe0)
    u0 = _sk_l0(y0, z0, src_r, dst_r)

    y1, a1, xs1 = pl.pallas_call(
        _tc1_body,
        grid=(GRID,),
        in_specs=[
            _row_spec(2, 8, BN, 128),
            _row_spec(2, BN, 16),
            _row_spec(BN, 128),
            _full_spec(1, 1024),
            _full_spec(128, 512),
            _full_spec(1, 512),
            _full_spec(8, 64),
            _full_spec(128, 64),
            _full_spec(1, 64),
        ],
        out_specs=[
            _row_spec(4, BN, 128),
            _row_spec(BN, 16),
            _row_spec(BN, 64),
        ],
        out_shape=[
            jax.ShapeDtypeStruct((4, NP, 128), f32),
            jax.ShapeDtypeStruct((NP, 16), f32),
            jax.ShapeDtypeStruct((NP, 64), f32),
        ],
    )(u0, s0p, x0, We0, Wm1, b1, att1, Ws1, bs1.reshape(1, 64))

    z1, s1p = _zk16(a1, et, src_r, dst_r, awe1)
    u1 = _sk_l1(y1, z1, src_r, dst_r)

    y2, a2, x2 = pl.pallas_call(
        _tc2_body,
        grid=(GRID,),
        in_specs=[
            _row_spec(2, 4, BN, 128),
            _row_spec(2, BN, 16),
            _row_spec(BN, 64),
            _full_spec(1, 512),
            _full_spec(64, 1024),
            _full_spec(1, 1024),
            _full_spec(16, 64),
        ],
        out_specs=[
            _row_spec(8, BN, 128),
            _row_spec(BN, 16),
            _row_spec(BN, 64),
        ],
        out_shape=[
            jax.ShapeDtypeStruct((8, NP, 128), f32),
            jax.ShapeDtypeStruct((NP, 16), f32),
            jax.ShapeDtypeStruct((NP, 64), f32),
        ],
    )(u1, s1p, xs1, We1, Wm2, b2, att2)

    z2, s2p = _zk32(a2, et, src_r, dst_r, awe2)
    u2 = _sk_l2(y2, z2, src_r, dst_r)

    y3, a3, xs3 = pl.pallas_call(
        _tc3_body,
        grid=(GRID,),
        in_specs=[
            _row_spec(2, 8, BN, 128),
            _row_spec(2, BN, 32),
            _row_spec(BN, 64),
            _full_spec(1, 1024),
            _full_spec(64, 32),
            _full_spec(1, 32),
            _full_spec(1, 32),
            _full_spec(64, 32),
            _full_spec(1, 32),
        ],
        out_specs=[
            _row_spec(BN, 32),
            _row_spec(BN, 16),
            _row_spec(BN, 32),
        ],
        out_shape=[
            jax.ShapeDtypeStruct((NP, 32), f32),
            jax.ShapeDtypeStruct((NP, 16), f32),
            jax.ShapeDtypeStruct((NP, 32), f32),
        ],
    )(u2, s2p, x2, We2, Wm3, bm3.reshape(1, 32), att3, Ws3,
      bs3.reshape(1, 32))

    u3, s3p = _l3_kernel(a3, y3, src_r, dst_r)

    critic, h = pl.pallas_call(
        _tc4_body,
        grid=(GRID,),
        in_specs=[
            _row_spec(2, BN, 32),
            _row_spec(2, BN, 16),
            _row_spec(BN, 32),
            _full_spec(32, 1),
            _full_spec(1, 1),
        ],
        out_specs=[
            _row_spec(BN, 1),
            _row_spec(BN, 32),
        ],
        out_shape=[
            jax.ShapeDtypeStruct((NP, 1), f32),
            jax.ShapeDtypeStruct((NP, 32), f32),
        ],
    )(u3, s3p, xs3, Wc, bc.reshape(1, 1))

    return (critic[:N], h[:N])


# speedup vs baseline: 1.7714x; 1.7714x over previous
import functools

import jax
import jax.numpy as jnp
from jax import lax
from jax.experimental import pallas as pl
from jax.experimental.pallas import tpu as pltpu
from jax.experimental.pallas import tpu_sc as plsc

N = 10000
NP = 10240
E = 320000
NC, NS = 2, 16
NT = NC * NS
EPT = E // NT
SUB = 100
B = 200
NSUB = B // SUB
NBLK = EPT // B
RPT = NP // NS
ZR = 64
BN = 1024
GRID = NP // BN

f32 = jnp.float32
i32 = jnp.int32

_MESH = plsc.VectorSubcoreMesh(core_axis_name="c", subcore_axis_name="s")


def _b16(v):
    return jnp.full((16,), v, i32)


def _worker(cid, sid):
    return sid * NC + cid


def _make_zkernel(SW):
    dup = SW == 16

    @functools.partial(
        pl.kernel,
        out_type=(
            jax.ShapeDtypeStruct((SW, E), f32),
            jax.ShapeDtypeStruct((NC, NP, SW), f32),
        ),
        mesh=_MESH,
        compiler_params=pltpu.CompilerParams(needs_layout_passes=False, use_tc_tiling_on_sc=False),
        scratch_types=[
            pltpu.VMEM((NSUB, SUB), i32),
            pltpu.VMEM((NSUB, SUB), i32),
            pltpu.VMEM((B,), f32),
            pltpu.VMEM((B, 16), f32),
            pltpu.VMEM((B, SW), f32),
            pltpu.VMEM((SW, B), f32),
            pltpu.VMEM((16,), f32),
            pltpu.VMEM((ZR, SW), f32),
            pltpu.VMEM_SHARED((NP, SW), f32),
            pltpu.SemaphoreType.DMA,
        ],
    )
    def zkernel(a_hbm, et_hbm, srcr, dstr, awe_hbm, z_hbm, s_hbm,
                src_v, dst_v, et_v, a_rows, srows, srows_t, awe_v, zero_v,
                s_slab, sem):
        cid = lax.axis_index("c")
        sid = lax.axis_index("s")
        wid = _worker(cid, sid)
        pltpu.sync_copy(awe_hbm, awe_v)

        @pl.loop(0, ZR)
        def _zero(r):
            for c in range(SW // 16):
                zero_v[r, pl.ds(c * 16, 16)] = jnp.zeros((16,), f32)

        for i in range(RPT // ZR):
            pltpu.sync_copy(zero_v, s_slab.at[pl.ds(sid * RPT + i * ZR, ZR)])
        plsc.subcore_barrier()

        lane = lax.iota(i32, 16)
        awe = awe_v[...]

        @pl.loop(0, NBLK)
        def _blk(b):
            e0 = wid * EPT + b * B
            pltpu.sync_copy(srcr.at[wid, b], src_v)
            pltpu.sync_copy(dstr.at[wid, b], dst_v)
            pltpu.sync_copy(et_hbm.at[pl.ds(e0, B)], et_v)
            cps = [
                pltpu.async_copy(a_hbm.at[src_v.at[j]],
                                 a_rows.at[pl.ds(j * SUB, SUB)], sem)
                for j in range(NSUB)
            ]
            for cp in cps:
                cp.wait()

            @plsc.parallel_loop(0, B, unroll=8)
            def _edge(e):
                a16 = a_rows[e, :]
                etb = plsc.load_gather(et_v, [_b16(e)])
                lg = a16 + etb * awe
                lg = jnp.where(lg < 0.0, lg * 0.2, lg)
                z16 = jnp.exp(lg)
                if dup:
                    srow = jnp.where(lane < 8, z16, z16 * etb)
                    srows[e, :] = srow
                    plsc.store_scatter(srows_t, [lane, _b16(e)], srow)
                else:
                    zet = z16 * etb
                    srows[e, pl.ds(0, 16)] = z16
                    srows[e, pl.ds(16, 16)] = zet
                    plsc.store_scatter(srows_t, [lane, _b16(e)], z16)
                    plsc.store_scatter(srows_t, [lane + 16, _b16(e)], zet)

            for j in range(NSUB):
                pltpu.sync_copy(srows.at[pl.ds(j * SUB, SUB)],
                                s_slab.at[dst_v.at[j]], add=True)
            pltpu.sync_copy(srows_t, z_hbm.at[:, pl.ds(e0, B)])

        plsc.subcore_barrier()
        pltpu.sync_copy(s_slab.at[pl.ds(sid * RPT, RPT)],
                        s_hbm.at[cid, pl.ds(sid * RPT, RPT)])

    return zkernel


def _make_scale_kernel(K, hpc, SW):
    W = 128
    BS = 80
    NB = EPT // BS
    R = 3 if hpc == 1 else 2
    ZRS = 32
    del SW

    @functools.partial(
        pl.kernel,
        out_type=jax.ShapeDtypeStruct((NC, K, NP, W), f32),
        mesh=_MESH,
        compiler_params=pltpu.CompilerParams(needs_layout_passes=False, use_tc_tiling_on_sc=False),
        scratch_types=[
            pltpu.VMEM((4, BS), i32),
            pltpu.VMEM((4, BS), i32),
            pltpu.VMEM((hpc, EPT), f32),
            pltpu.VMEM((R, BS, W), f32),
            pltpu.VMEM((ZRS, W), f32),
            pltpu.VMEM_SHARED((NP, W), f32),
            pltpu.SemaphoreType.DMA,
            pltpu.SemaphoreType.DMA,
            pltpu.SemaphoreType.DMA,
            pltpu.SemaphoreType.DMA,
        ],
    )
    def skernel(y_hbm, zt_hbm, srcr, dstr, u_hbm,
                src_v, dst_v, zres, rows_v, zero_v, u_slab,
                gsem, ssem, isem, zsem):
        cid = lax.axis_index("c")
        sid = lax.axis_index("s")
        wid = _worker(cid, sid)

        @pl.loop(0, ZRS)
        def _zero(r):
            for c in range(W // 16):
                zero_v[r, pl.ds(c * 16, 16)] = jnp.zeros((16,), f32)

        def scale(k, b, p):
            @plsc.parallel_loop(0, BS, step=10)
            def _grp(eg):
                for jj in range(10):
                    e = eg + jj
                    ge = b * BS + e
                    if hpc == 1:
                        zb = plsc.load_gather(zres, [_b16(0), _b16(ge)])
                        for c in range(8):
                            rows_v[p, e, pl.ds(c * 16, 16)] = (
                                rows_v[p, e, pl.ds(c * 16, 16)] * zb)
                    else:
                        zb0 = plsc.load_gather(zres, [_b16(0), _b16(ge)])
                        zb1 = plsc.load_gather(zres, [_b16(1), _b16(ge)])
                        for c in range(4):
                            rows_v[p, e, pl.ds(c * 16, 16)] = (
                                rows_v[p, e, pl.ds(c * 16, 16)] * zb0)
                        for c in range(4, 8):
                            rows_v[p, e, pl.ds(c * 16, 16)] = (
                                rows_v[p, e, pl.ds(c * 16, 16)] * zb1)

        @pl.loop(0, K)
        def _chunk(k):
            zcps = [pltpu.async_copy(
                zero_v, u_slab.at[pl.ds(sid * RPT + i * ZRS, ZRS)], zsem)
                for i in range(RPT // ZRS)]
            for hh in range(hpc):
                pltpu.sync_copy(
                    zt_hbm.at[hpc * k + hh, pl.ds(wid * EPT, EPT)],
                    zres.at[hh])
            for cp in zcps:
                cp.wait()
            plsc.subcore_barrier()

            pltpu.sync_copy(srcr.at[wid, 0], src_v.at[0])
            pltpu.sync_copy(dstr.at[wid, 0], dst_v.at[0])
            pltpu.async_copy(y_hbm.at[k].at[src_v.at[0]], rows_v.at[0],
                             gsem)
            pltpu.async_copy(srcr.at[wid, 1], src_v.at[1], isem)
            pltpu.async_copy(dstr.at[wid, 1], dst_v.at[1], isem)

            @pl.loop(0, NB)
            def _blk(b):
                p = lax.rem(b, R)
                q = lax.rem(b + 1, R)
                i1m = lax.rem(b + 1, 4)
                i2m = lax.rem(b + 2, 4)

                @pl.when(b >= R - 1)
                def _wait_scatter():
                    jb = lax.rem(b + 1 - R, 4)
                    pltpu.make_async_copy(
                        rows_v.at[q], u_slab.at[dst_v.at[jb]], ssem).wait()

                @pl.when(b + 1 < NB)
                def _fire_gather():
                    pltpu.make_async_copy(
                        srcr.at[wid, b + 1], src_v.at[i1m], isem).wait()
                    pltpu.make_async_copy(
                        dstr.at[wid, b + 1], dst_v.at[i1m], isem).wait()
                    pltpu.async_copy(y_hbm.at[k].at[src_v.at[i1m]],
                                     rows_v.at[q], gsem)

                @pl.when(b + 2 < NB)
                def _prefetch_idx():
                    pltpu.async_copy(srcr.at[wid, b + 2], src_v.at[i2m],
                                     isem)
                    pltpu.async_copy(dstr.at[wid, b + 2], dst_v.at[i2m],
                                     isem)

                pltpu.make_async_copy(
                    y_hbm.at[k].at[src_v.at[lax.rem(b, 4)]],
                    rows_v.at[p], gsem).wait()
                scale(k, b, p)
                pltpu.async_copy(rows_v.at[p],
                                 u_slab.at[dst_v.at[lax.rem(b, 4)]],
                                 ssem, add=True)

            for d in range(1, R):
                j = NB - R + d
                pltpu.make_async_copy(
                    rows_v.at[j % R],
                    u_slab.at[dst_v.at[j % 4]], ssem).wait()
            plsc.subcore_barrier()
            pltpu.sync_copy(u_slab.at[pl.ds(sid * RPT, RPT)],
                            u_hbm.at[cid, k, pl.ds(sid * RPT, RPT)])
            plsc.subcore_barrier()

    return skernel


@functools.partial(
    pl.kernel,
    out_type=(
        jax.ShapeDtypeStruct((NC, NP, 32), f32),
        jax.ShapeDtypeStruct((NC, NP, 16), f32),
    ),
    mesh=_MESH,
    compiler_params=pltpu.CompilerParams(needs_layout_passes=False, use_tc_tiling_on_sc=False),
    scratch_types=[
        pltpu.VMEM((NSUB, SUB), i32),
        pltpu.VMEM((NSUB, SUB), i32),
        pltpu.VMEM((B, 16), f32),
        pltpu.VMEM((B, 32), f32),
        pltpu.VMEM((B, 16), f32),
        pltpu.VMEM((ZR, 32), f32),
        pltpu.VMEM_SHARED((NP, 32), f32),
        pltpu.VMEM_SHARED((NP, 16), f32),
        pltpu.SemaphoreType.DMA,
    ],
)
def _l3_kernel(a_hbm, y_hbm, srcr, dstr, u_hbm, s_hbm,
               src_v, dst_v, a_rows, rows_v, srows, zero_v, u_slab, s_slab,
               sem):
    cid = lax.axis_index("c")
    sid = lax.axis_index("s")
    wid = _worker(cid, sid)

    @pl.loop(0, ZR)
    def _zero(r):
        for c in range(2):
            zero_v[r, pl.ds(c * 16, 16)] = jnp.zeros((16,), f32)

    for i in range(RPT // ZR):
        pltpu.sync_copy(zero_v, u_slab.at[pl.ds(sid * RPT + i * ZR, ZR)])
        pltpu.sync_copy(zero_v.at[:, pl.ds(0, 16)],
                        s_slab.at[pl.ds(sid * RPT + i * ZR, ZR)])
    plsc.subcore_barrier()

    lane = lax.iota(i32, 16)

    @pl.loop(0, NBLK)
    def _blk(b):
        pltpu.sync_copy(srcr.at[wid, b], src_v)
        pltpu.sync_copy(dstr.at[wid, b], dst_v)
        cps = [
            pltpu.async_copy(a_hbm.at[src_v.at[j]],
                             a_rows.at[pl.ds(j * SUB, SUB)], sem)
            for j in range(NSUB)
        ] + [
            pltpu.async_copy(y_hbm.at[src_v.at[j]],
                             rows_v.at[pl.ds(j * SUB, SUB)], sem)
            for j in range(NSUB)
        ]
        for cp in cps:
            cp.wait()

        @plsc.parallel_loop(0, B, unroll=8)
        def _edge(e):
            a16 = a_rows[e, :]
            lg = jnp.where(a16 < 0.0, a16 * 0.2, a16)
            z16 = jnp.exp(lg)
            srows[e, :] = jnp.where(lane < 1, z16, jnp.zeros((16,), f32))
            rows_v[e, pl.ds(0, 16)] = rows_v[e, pl.ds(0, 16)] * z16
            rows_v[e, pl.ds(16, 16)] = rows_v[e, pl.ds(16, 16)] * z16

        for j in range(NSUB):
            pltpu.sync_copy(rows_v.at[pl.ds(j * SUB, SUB)],
                            u_slab.at[dst_v.at[j]], add=True)
            pltpu.sync_copy(srows.at[pl.ds(j * SUB, SUB)],
                            s_slab.at[dst_v.at[j]], add=True)

    plsc.subcore_barrier()
    pltpu.sync_copy(u_slab.at[pl.ds(sid * RPT, RPT)],
                    u_hbm.at[cid, pl.ds(sid * RPT, RPT)])
    pltpu.sync_copy(s_slab.at[pl.ds(sid * RPT, RPT)],
                    s_hbm.at[cid, pl.ds(sid * RPT, RPT)])


def _elu(x):
    return jnp.where(x > 0.0, x, jnp.exp(x) - 1.0)


def _combine(u, s, We, heads, C, eps=1e-16):
    s0 = s[:, :heads]
    acc = jnp.zeros((u.shape[1], C), f32)
    for h in range(heads):
        if C == 128:
            uh = u[h]
        else:
            k, half = h // 2, h % 2
            uh = u[k][:, half * 64:half * 64 + 64]
        if We is not None:
            s1 = s[:, heads:]
            uh = uh + s1[:, h:h + 1] * We[0, h * C:(h + 1) * C][None, :]
        acc = acc + uh / (s0[:, h:h + 1] + eps)
    return acc / heads


def _tc0_body(nodes_ref, emb_ref, Wm0_ref, b0_ref, att0_ref,
              y0_ref, a0_ref, x0_ref):
    nodes = nodes_ref[0, 0, :]
    oh = (nodes[:, None] == lax.broadcasted_iota(i32, (BN, 50), 1))
    xb = jnp.dot(oh.astype(f32), emb_ref[...],
                 preferred_element_type=f32)
    y = jnp.dot(xb, Wm0_ref[...], preferred_element_type=f32) + b0_ref[0]
    for k in range(8):
        y0_ref[k] = y[:, k * 128:(k + 1) * 128]
    an = jnp.sum(y.reshape(BN, 8, 128) * att0_ref[...][None], axis=-1)
    a0_ref[...] = jnp.concatenate([an, an], axis=1)
    x0_ref[...] = xb


def _tc1_body(u_ref, s_ref, x0_ref, We0_ref, Wm1_ref, b1_ref, att1_ref,
              Ws1_ref, bs1_ref, y1_ref, a1_ref, xs1_ref):
    u = u_ref[0] + u_ref[1]
    s = s_ref[0] + s_ref[1]
    out = _combine(u, s, We0_ref[...], 8, 128)
    x1 = _elu(out + x0_ref[...])
    y = jnp.dot(x1, Wm1_ref[...], preferred_element_type=f32) + b1_ref[0]
    for k in range(4):
        y1_ref[k] = y[:, k * 128:(k + 1) * 128]
    an = jnp.sum(y.reshape(BN, 8, 64) * att1_ref[...][None], axis=-1)
    a1_ref[...] = jnp.concatenate([an, an], axis=1)
    xs1_ref[...] = jnp.dot(x1, Ws1_ref[...],
                           preferred_element_type=f32) + bs1_ref[0]


def _tc2_body(u_ref, s_ref, xs1_ref, We1_ref, Wm2_ref, b2_ref, att2_ref,
              y2_ref, a2_ref, x2_ref):
    u = u_ref[0] + u_ref[1]
    s = s_ref[0] + s_ref[1]
    out = _combine(u, s, We1_ref[...], 8, 64)
    x2 = _elu(out + xs1_ref[...])
    y = jnp.dot(x2, Wm2_ref[...], preferred_element_type=f32) + b2_ref[0]
    for k in range(8):
        y2_ref[k] = y[:, k * 128:(k + 1) * 128]
    a2_ref[...] = jnp.sum(y.reshape(BN, 16, 64) * att2_ref[...][None],
                          axis=-1)
    x2_ref[...] = x2


def _tc3_body(u_ref, s_ref, x2_ref, We2_ref, Wm3_ref, bm3_ref, att3_ref,
              Ws3_ref, bs3_ref, y3_ref, a3_ref, xs3_ref):
    u = u_ref[0] + u_ref[1]
    s = s_ref[0] + s_ref[1]
    out = _combine(u, s, We2_ref[...], 16, 64)
    x3 = _elu(out + x2_ref[...])
    y = jnp.dot(x3, Wm3_ref[...], preferred_element_type=f32) + bm3_ref[0]
    y3_ref[...] = y
    an = jnp.sum(y.reshape(BN, 1, 32) * att3_ref[...][None], axis=-1)
    a3_ref[...] = jnp.broadcast_to(an, (BN, 16))
    xs3_ref[...] = jnp.dot(x3, Ws3_ref[...],
                           preferred_element_type=f32) + bs3_ref[0]


def _tc4_body(u_ref, s_ref, xs3_ref, Wc_ref, bc_ref, crit_ref, h_ref):
    u = u_ref[0] + u_ref[1]
    s0 = s_ref[0, :, 0:1] + s_ref[1, :, 0:1]
    h = u / (s0 + 1e-16) + xs3_ref[...]
    h_ref[...] = h
    crit_ref[...] = jnp.dot(h, Wc_ref[...],
                            preferred_element_type=f32) + bc_ref[0]


def _row_spec(*shape):
    nd = len(shape)

    def im(i):
        return tuple(0 for _ in range(nd - 2)) + (i, 0)

    return pl.BlockSpec(shape, im)


def _full_spec(*shape):
    nd = len(shape)

    def im(i):
        return tuple(0 for _ in range(nd))

    return pl.BlockSpec(shape, im)


_zk16 = _make_zkernel(16)
_zk32 = _make_zkernel(32)
_sk_l0 = _make_scale_kernel(8, 1, 16)
_sk_l1 = _make_scale_kernel(4, 2, 16)
_sk_l2 = _make_scale_kernel(8, 2, 32)


def kernel(nodes, edges, edge_type, emb, Wm0, bm0, We0, be0, att0, Wm1, bm1,
           We1, be1, att1, Ws1, bs1, Wm2, bm2, We2, be2, att2, Wm3, bm3,
           att3, Ws3, bs3, Wc, bc):
    src_r = edges[0].astype(i32).reshape(NT, NBLK, NSUB, SUB)
    dst_r = edges[1].astype(i32).reshape(NT, NBLK, NSUB, SUB)
    src_s = edges[0].astype(i32).reshape(NT, EPT // 80, 80)
    dst_s = edges[1].astype(i32).reshape(NT, EPT // 80, 80)
    et = edge_type.astype(f32)

    b0 = (bm0 + be0).reshape(1, 1024)
    b1 = (bm1 + be1).reshape(1, 512)
    b2 = (bm2 + be2).reshape(1, 1024)
    awe0 = jnp.sum(We0.reshape(8, 128) * att0, axis=-1)
    awe0 = jnp.concatenate([awe0, awe0])
    awe1 = jnp.sum(We1.reshape(8, 64) * att1, axis=-1)
    awe1 = jnp.concatenate([awe1, awe1])
    awe2 = jnp.sum(We2.reshape(16, 64) * att2, axis=-1)
    nodes3 = jnp.pad(nodes.astype(i32), (0, NP - N)).reshape(GRID, 1, BN)

    y0, a0, x0 = pl.pallas_call(
        _tc0_body,
        grid=(GRID,),
        in_specs=[
            pl.BlockSpec((1, 1, BN), lambda i: (i, 0, 0)),
            _full_spec(50, 128),
            _full_spec(128, 1024),
            _full_spec(1, 1024),
            _full_spec(8, 128),
        ],
        out_specs=[
            _row_spec(8, BN, 128),
            _row_spec(BN, 16),
            _row_spec(BN, 128),
        ],
        out_shape=[
            jax.ShapeDtypeStruct((8, NP, 128), f32),
            jax.ShapeDtypeStruct((NP, 16), f32),
            jax.ShapeDtypeStruct((NP, 128), f32),
        ],
    )(nodes3, emb, Wm0, b0, att0)

    z0, s0p = _zk16(a0, et, src_r, dst_r, awe0)
    u0 = _sk_l0(y0, z0, src_s, dst_s)

    y1, a1, xs1 = pl.pallas_call(
        _tc1_body,
        grid=(GRID,),
        in_specs=[
            _row_spec(2, 8, BN, 128),
            _row_spec(2, BN, 16),
            _row_spec(BN, 128),
            _full_spec(1, 1024),
            _full_spec(128, 512),
            _full_spec(1, 512),
            _full_spec(8, 64),
            _full_spec(128, 64),
            _full_spec(1, 64),
        ],
        out_specs=[
            _row_spec(4, BN, 128),
            _row_spec(BN, 16),
            _row_spec(BN, 64),
        ],
        out_shape=[
            jax.ShapeDtypeStruct((4, NP, 128), f32),
            jax.ShapeDtypeStruct((NP, 16), f32),
            jax.ShapeDtypeStruct((NP, 64), f32),
        ],
    )(u0, s0p, x0, We0, Wm1, b1, att1, Ws1, bs1.reshape(1, 64))

    z1, s1p = _zk16(a1, et, src_r, dst_r, awe1)
    u1 = _sk_l1(y1, z1, src_s, dst_s)

    y2, a2, x2 = pl.pallas_call(
        _tc2_body,
        grid=(GRID,),
        in_specs=[
            _row_spec(2, 4, BN, 128),
            _row_spec(2, BN, 16),
            _row_spec(BN, 64),
            _full_spec(1, 512),
            _full_spec(64, 1024),
            _full_spec(1, 1024),
            _full_spec(16, 64),
        ],
        out_specs=[
            _row_spec(8, BN, 128),
            _row_spec(BN, 16),
            _row_spec(BN, 64),
        ],
        out_shape=[
            jax.ShapeDtypeStruct((8, NP, 128), f32),
            jax.ShapeDtypeStruct((NP, 16), f32),
            jax.ShapeDtypeStruct((NP, 64), f32),
        ],
    )(u1, s1p, xs1, We1, Wm2, b2, att2)

    z2, s2p = _zk32(a2, et, src_r, dst_r, awe2)
    u2 = _sk_l2(y2, z2, src_s, dst_s)

    y3, a3, xs3 = pl.pallas_call(
        _tc3_body,
        grid=(GRID,),
        in_specs=[
            _row_spec(2, 8, BN, 128),
            _row_spec(2, BN, 32),
            _row_spec(BN, 64),
            _full_spec(1, 1024),
            _full_spec(64, 32),
            _full_spec(1, 32),
            _full_spec(1, 32),
            _full_spec(64, 32),
            _full_spec(1, 32),
        ],
        out_specs=[
            _row_spec(BN, 32),
            _row_spec(BN, 16),
            _row_spec(BN, 32),
        ],
        out_shape=[
            jax.ShapeDtypeStruct((NP, 32), f32),
            jax.ShapeDtypeStruct((NP, 16), f32),
            jax.ShapeDtypeStruct((NP, 32), f32),
        ],
    )(u2, s2p, x2, We2, Wm3, bm3.reshape(1, 32), att3, Ws3,
      bs3.reshape(1, 32))

    u3, s3p = _l3_kernel(a3, y3, src_r, dst_r)

    critic, h = pl.pallas_call(
        _tc4_body,
        grid=(GRID,),
        in_specs=[
            _row_spec(2, BN, 32),
            _row_spec(2, BN, 16),
            _row_spec(BN, 32),
            _full_spec(32, 1),
            _full_spec(1, 1),
        ],
        out_specs=[
            _row_spec(BN, 1),
            _row_spec(BN, 32),
        ],
        out_shape=[
            jax.ShapeDtypeStruct((NP, 1), f32),
            jax.ShapeDtypeStruct((NP, 32), f32),
        ],
    )(u3, s3p, xs3, Wc, bc.reshape(1, 1))

    return (critic[:N], h[:N])


# speedup vs baseline: 1.8300x; 1.0331x over previous
import functools

import jax
import jax.numpy as jnp
from jax import lax
from jax.experimental import pallas as pl
from jax.experimental.pallas import tpu as pltpu
from jax.experimental.pallas import tpu_sc as plsc

N = 10000
NP = 10240
E = 320000
NC, NS = 2, 16
NT = NC * NS
EPT = E // NT
SUB = 100
B = 200
NSUB = B // SUB
NBLK = EPT // B
RPT = NP // NS
ZR = 64
BN = 1024
GRID = NP // BN

f32 = jnp.float32
i32 = jnp.int32

_MESH = plsc.VectorSubcoreMesh(core_axis_name="c", subcore_axis_name="s")


def _b16(v):
    return jnp.full((16,), v, i32)


def _worker(cid, sid):
    return sid * NC + cid


def _make_zkernel(SW):
    dup = SW == 16

    @functools.partial(
        pl.kernel,
        out_type=(
            jax.ShapeDtypeStruct((SW, E), f32),
            jax.ShapeDtypeStruct((NC, NP, SW), f32),
        ),
        mesh=_MESH,
        compiler_params=pltpu.CompilerParams(needs_layout_passes=False, use_tc_tiling_on_sc=False),
        scratch_types=[
            pltpu.VMEM((2, NSUB, SUB), i32),
            pltpu.VMEM((2, NSUB, SUB), i32),
            pltpu.VMEM((2, B), f32),
            pltpu.VMEM((2, B, 16), f32),
            pltpu.VMEM((B, SW), f32),
            pltpu.VMEM((SW, B), f32),
            pltpu.VMEM((16,), f32),
            pltpu.VMEM((ZR, SW), f32),
            pltpu.VMEM_SHARED((NP, SW), f32),
            pltpu.SemaphoreType.DMA,
        ],
    )
    def zkernel(a_hbm, et_hbm, srcr, dstr, awe_hbm, z_hbm, s_hbm,
                src_v, dst_v, et_v, a_rows, srows, srows_t, awe_v, zero_v,
                s_slab, sem):
        cid = lax.axis_index("c")
        sid = lax.axis_index("s")
        wid = _worker(cid, sid)
        pltpu.sync_copy(awe_hbm, awe_v)

        @pl.loop(0, ZR)
        def _zero(r):
            for c in range(SW // 16):
                zero_v[r, pl.ds(c * 16, 16)] = jnp.zeros((16,), f32)

        for i in range(RPT // ZR):
            pltpu.sync_copy(zero_v, s_slab.at[pl.ds(sid * RPT + i * ZR, ZR)])
        plsc.subcore_barrier()

        lane = lax.iota(i32, 16)
        awe = awe_v[...]

        def load_blk(b, r):
            e0 = wid * EPT + b * B
            pltpu.sync_copy(srcr.at[wid, b], src_v.at[r])
            pltpu.sync_copy(dstr.at[wid, b], dst_v.at[r])
            pltpu.sync_copy(et_hbm.at[pl.ds(e0, B)], et_v.at[r])
            for j in range(NSUB):
                pltpu.async_copy(a_hbm.at[src_v.at[r, j]],
                                 a_rows.at[r, pl.ds(j * SUB, SUB)], sem)

        load_blk(0, 0)

        @pl.loop(0, NBLK // 2)
        def _pair(g):
            for p in range(2):
                b = g * 2 + p
                q = 1 - p

                @pl.when(b + 1 < NBLK)
                def _pref():
                    load_blk(b + 1, q)

                for j in range(NSUB):
                    pltpu.make_async_copy(
                        a_hbm.at[src_v.at[p, j]],
                        a_rows.at[p, pl.ds(j * SUB, SUB)], sem).wait()

                @plsc.parallel_loop(0, B, unroll=8)
                def _edge(e):
                    a16 = a_rows[p, e, :]
                    etb = plsc.load_gather(et_v, [_b16(p), _b16(e)])
                    lg = a16 + etb * awe
                    lg = jnp.where(lg < 0.0, lg * 0.2, lg)
                    z16 = jnp.exp(lg)
                    if dup:
                        srow = jnp.where(lane < 8, z16, z16 * etb)
                        srows[e, :] = srow
                        plsc.store_scatter(srows_t, [lane, _b16(e)], srow)
                    else:
                        zet = z16 * etb
                        srows[e, pl.ds(0, 16)] = z16
                        srows[e, pl.ds(16, 16)] = zet
                        plsc.store_scatter(srows_t, [lane, _b16(e)], z16)
                        plsc.store_scatter(srows_t, [lane + 16, _b16(e)],
                                           zet)

                e0 = wid * EPT + b * B
                for j in range(NSUB):
                    pltpu.sync_copy(srows.at[pl.ds(j * SUB, SUB)],
                                    s_slab.at[dst_v.at[p, j]], add=True)
                pltpu.sync_copy(srows_t, z_hbm.at[:, pl.ds(e0, B)])

        plsc.subcore_barrier()
        pltpu.sync_copy(s_slab.at[pl.ds(sid * RPT, RPT)],
                        s_hbm.at[cid, pl.ds(sid * RPT, RPT)])

    return zkernel


def _make_scale_kernel(K, hpc, SW):
    W = 128
    BS = 80
    NB = EPT // BS
    R = 3 if hpc == 1 else 2
    ZRS = 32
    del SW

    @functools.partial(
        pl.kernel,
        out_type=jax.ShapeDtypeStruct((NC, K, NP, W), f32),
        mesh=_MESH,
        compiler_params=pltpu.CompilerParams(needs_layout_passes=False, use_tc_tiling_on_sc=False),
        scratch_types=[
            pltpu.VMEM((4, BS), i32),
            pltpu.VMEM((4, BS), i32),
            pltpu.VMEM((hpc, EPT), f32),
            pltpu.VMEM((R, BS, W), f32),
            pltpu.VMEM((ZRS, W), f32),
            pltpu.VMEM_SHARED((NP, W), f32),
            pltpu.SemaphoreType.DMA,
            pltpu.SemaphoreType.DMA,
            pltpu.SemaphoreType.DMA,
            pltpu.SemaphoreType.DMA,
        ],
    )
    def skernel(y_hbm, zt_hbm, srcr, dstr, u_hbm,
                src_v, dst_v, zres, rows_v, zero_v, u_slab,
                gsem, ssem, isem, zsem):
        cid = lax.axis_index("c")
        sid = lax.axis_index("s")
        wid = _worker(cid, sid)

        @pl.loop(0, ZRS)
        def _zero(r):
            for c in range(W // 16):
                zero_v[r, pl.ds(c * 16, 16)] = jnp.zeros((16,), f32)

        def scale(k, b, p):
            @plsc.parallel_loop(0, BS, step=10)
            def _grp(eg):
                for jj in range(10):
                    e = eg + jj
                    ge = b * BS + e
                    if hpc == 1:
                        zb = plsc.load_gather(zres, [_b16(0), _b16(ge)])
                        for c in range(8):
                            rows_v[p, e, pl.ds(c * 16, 16)] = (
                                rows_v[p, e, pl.ds(c * 16, 16)] * zb)
                    else:
                        zb0 = plsc.load_gather(zres, [_b16(0), _b16(ge)])
                        zb1 = plsc.load_gather(zres, [_b16(1), _b16(ge)])
                        for c in range(4):
                            rows_v[p, e, pl.ds(c * 16, 16)] = (
                                rows_v[p, e, pl.ds(c * 16, 16)] * zb0)
                        for c in range(4, 8):
                            rows_v[p, e, pl.ds(c * 16, 16)] = (
                                rows_v[p, e, pl.ds(c * 16, 16)] * zb1)

        @pl.loop(0, K)
        def _chunk(k):
            zcps = [pltpu.async_copy(
                zero_v, u_slab.at[pl.ds(sid * RPT + i * ZRS, ZRS)], zsem)
                for i in range(RPT // ZRS)]
            for hh in range(hpc):
                pltpu.sync_copy(
                    zt_hbm.at[hpc * k + hh, pl.ds(wid * EPT, EPT)],
                    zres.at[hh])
            for cp in zcps:
                cp.wait()
            plsc.subcore_barrier()

            pltpu.sync_copy(srcr.at[wid, 0], src_v.at[0])
            pltpu.sync_copy(dstr.at[wid, 0], dst_v.at[0])
            pltpu.async_copy(y_hbm.at[k].at[src_v.at[0]], rows_v.at[0],
                             gsem)
            pltpu.async_copy(srcr.at[wid, 1], src_v.at[1], isem)
            pltpu.async_copy(dstr.at[wid, 1], dst_v.at[1], isem)

            @pl.loop(0, NB)
            def _blk(b):
                p = lax.rem(b, R)
                q = lax.rem(b + 1, R)
                i1m = lax.rem(b + 1, 4)
                i2m = lax.rem(b + 2, 4)

                @pl.when(b >= R - 1)
                def _wait_scatter():
                    jb = lax.rem(b + 1 - R, 4)
                    pltpu.make_async_copy(
                        rows_v.at[q], u_slab.at[dst_v.at[jb]], ssem).wait()

                @pl.when(b + 1 < NB)
                def _fire_gather():
                    pltpu.make_async_copy(
                        srcr.at[wid, b + 1], src_v.at[i1m], isem).wait()
                    pltpu.make_async_copy(
                        dstr.at[wid, b + 1], dst_v.at[i1m], isem).wait()
                    pltpu.async_copy(y_hbm.at[k].at[src_v.at[i1m]],
                                     rows_v.at[q], gsem)

                @pl.when(b + 2 < NB)
                def _prefetch_idx():
                    pltpu.async_copy(srcr.at[wid, b + 2], src_v.at[i2m],
                                     isem)
                    pltpu.async_copy(dstr.at[wid, b + 2], dst_v.at[i2m],
                                     isem)

                pltpu.make_async_copy(
                    y_hbm.at[k].at[src_v.at[lax.rem(b, 4)]],
                    rows_v.at[p], gsem).wait()
                scale(k, b, p)
                pltpu.async_copy(rows_v.at[p],
                                 u_slab.at[dst_v.at[lax.rem(b, 4)]],
                                 ssem, add=True)

            for d in range(1, R):
                j = NB - R + d
                pltpu.make_async_copy(
                    rows_v.at[j % R],
                    u_slab.at[dst_v.at[j % 4]], ssem).wait()
            plsc.subcore_barrier()
            pltpu.sync_copy(u_slab.at[pl.ds(sid * RPT, RPT)],
                            u_hbm.at[cid, k, pl.ds(sid * RPT, RPT)])
            plsc.subcore_barrier()

    return skernel


@functools.partial(
    pl.kernel,
    out_type=(
        jax.ShapeDtypeStruct((NC, NP, 32), f32),
        jax.ShapeDtypeStruct((NC, NP, 16), f32),
    ),
    mesh=_MESH,
    compiler_params=pltpu.CompilerParams(needs_layout_passes=False, use_tc_tiling_on_sc=False),
    scratch_types=[
        pltpu.VMEM((NSUB, SUB), i32),
        pltpu.VMEM((NSUB, SUB), i32),
        pltpu.VMEM((B, 16), f32),
        pltpu.VMEM((B, 32), f32),
        pltpu.VMEM((B, 16), f32),
        pltpu.VMEM((ZR, 32), f32),
        pltpu.VMEM_SHARED((NP, 32), f32),
        pltpu.VMEM_SHARED((NP, 16), f32),
        pltpu.SemaphoreType.DMA,
    ],
)
def _l3_kernel(a_hbm, y_hbm, srcr, dstr, u_hbm, s_hbm,
               src_v, dst_v, a_rows, rows_v, srows, zero_v, u_slab, s_slab,
               sem):
    cid = lax.axis_index("c")
    sid = lax.axis_index("s")
    wid = _worker(cid, sid)

    @pl.loop(0, ZR)
    def _zero(r):
        for c in range(2):
            zero_v[r, pl.ds(c * 16, 16)] = jnp.zeros((16,), f32)

    for i in range(RPT // ZR):
        pltpu.sync_copy(zero_v, u_slab.at[pl.ds(sid * RPT + i * ZR, ZR)])
        pltpu.sync_copy(zero_v.at[:, pl.ds(0, 16)],
                        s_slab.at[pl.ds(sid * RPT + i * ZR, ZR)])
    plsc.subcore_barrier()

    lane = lax.iota(i32, 16)

    @pl.loop(0, NBLK)
    def _blk(b):
        pltpu.sync_copy(srcr.at[wid, b], src_v)
        pltpu.sync_copy(dstr.at[wid, b], dst_v)
        cps = [
            pltpu.async_copy(a_hbm.at[src_v.at[j]],
                             a_rows.at[pl.ds(j * SUB, SUB)], sem)
            for j in range(NSUB)
        ] + [
            pltpu.async_copy(y_hbm.at[src_v.at[j]],
                             rows_v.at[pl.ds(j * SUB, SUB)], sem)
            for j in range(NSUB)
        ]
        for cp in cps:
            cp.wait()

        @plsc.parallel_loop(0, B, unroll=8)
        def _edge(e):
            a16 = a_rows[e, :]
            lg = jnp.where(a16 < 0.0, a16 * 0.2, a16)
            z16 = jnp.exp(lg)
            srows[e, :] = jnp.where(lane < 1, z16, jnp.zeros((16,), f32))
            rows_v[e, pl.ds(0, 16)] = rows_v[e, pl.ds(0, 16)] * z16
            rows_v[e, pl.ds(16, 16)] = rows_v[e, pl.ds(16, 16)] * z16

        for j in range(NSUB):
            pltpu.sync_copy(rows_v.at[pl.ds(j * SUB, SUB)],
                            u_slab.at[dst_v.at[j]], add=True)
            pltpu.sync_copy(srows.at[pl.ds(j * SUB, SUB)],
                            s_slab.at[dst_v.at[j]], add=True)

    plsc.subcore_barrier()
    pltpu.sync_copy(u_slab.at[pl.ds(sid * RPT, RPT)],
                    u_hbm.at[cid, pl.ds(sid * RPT, RPT)])
    pltpu.sync_copy(s_slab.at[pl.ds(sid * RPT, RPT)],
                    s_hbm.at[cid, pl.ds(sid * RPT, RPT)])


def _elu(x):
    return jnp.where(x > 0.0, x, jnp.exp(x) - 1.0)


def _combine(u, s, We, heads, C, eps=1e-16):
    s0 = s[:, :heads]
    acc = jnp.zeros((u.shape[1], C), f32)
    for h in range(heads):
        if C == 128:
            uh = u[h]
        else:
            k, half = h // 2, h % 2
            uh = u[k][:, half * 64:half * 64 + 64]
        if We is not None:
            s1 = s[:, heads:]
            uh = uh + s1[:, h:h + 1] * We[0, h * C:(h + 1) * C][None, :]
        acc = acc + uh / (s0[:, h:h + 1] + eps)
    return acc / heads


def _tc0_body(nodes_ref, emb_ref, Wm0_ref, b0_ref, att0_ref,
              y0_ref, a0_ref, x0_ref):
    nodes = nodes_ref[0, 0, :]
    oh = (nodes[:, None] == lax.broadcasted_iota(i32, (BN, 50), 1))
    xb = jnp.dot(oh.astype(f32), emb_ref[...],
                 preferred_element_type=f32)
    y = jnp.dot(xb, Wm0_ref[...], preferred_element_type=f32) + b0_ref[0]
    for k in range(8):
        y0_ref[k] = y[:, k * 128:(k + 1) * 128]
    an = jnp.sum(y.reshape(BN, 8, 128) * att0_ref[...][None], axis=-1)
    a0_ref[...] = jnp.concatenate([an, an], axis=1)
    x0_ref[...] = xb


def _tc1_body(u_ref, s_ref, x0_ref, We0_ref, Wm1_ref, b1_ref, att1_ref,
              Ws1_ref, bs1_ref, y1_ref, a1_ref, xs1_ref):
    u = u_ref[0] + u_ref[1]
    s = s_ref[0] + s_ref[1]
    out = _combine(u, s, We0_ref[...], 8, 128)
    x1 = _elu(out + x0_ref[...])
    y = jnp.dot(x1, Wm1_ref[...], preferred_element_type=f32) + b1_ref[0]
    for k in range(4):
        y1_ref[k] = y[:, k * 128:(k + 1) * 128]
    an = jnp.sum(y.reshape(BN, 8, 64) * att1_ref[...][None], axis=-1)
    a1_ref[...] = jnp.concatenate([an, an], axis=1)
    xs1_ref[...] = jnp.dot(x1, Ws1_ref[...],
                           preferred_element_type=f32) + bs1_ref[0]


def _tc2_body(u_ref, s_ref, xs1_ref, We1_ref, Wm2_ref, b2_ref, att2_ref,
              y2_ref, a2_ref, x2_ref):
    u = u_ref[0] + u_ref[1]
    s = s_ref[0] + s_ref[1]
    out = _combine(u, s, We1_ref[...], 8, 64)
    x2 = _elu(out + xs1_ref[...])
    y = jnp.dot(x2, Wm2_ref[...], preferred_element_type=f32) + b2_ref[0]
    for k in range(8):
        y2_ref[k] = y[:, k * 128:(k + 1) * 128]
    a2_ref[...] = jnp.sum(y.reshape(BN, 16, 64) * att2_ref[...][None],
                          axis=-1)
    x2_ref[...] = x2


def _tc3_body(u_ref, s_ref, x2_ref, We2_ref, Wm3_ref, bm3_ref, att3_ref,
              Ws3_ref, bs3_ref, y3_ref, a3_ref, xs3_ref):
    u = u_ref[0] + u_ref[1]
    s = s_ref[0] + s_ref[1]
    out = _combine(u, s, We2_ref[...], 16, 64)
    x3 = _elu(out + x2_ref[...])
    y = jnp.dot(x3, Wm3_ref[...], preferred_element_type=f32) + bm3_ref[0]
    y3_ref[...] = y
    an = jnp.sum(y.reshape(BN, 1, 32) * att3_ref[...][None], axis=-1)
    a3_ref[...] = jnp.broadcast_to(an, (BN, 16))
    xs3_ref[...] = jnp.dot(x3, Ws3_ref[...],
                           preferred_element_type=f32) + bs3_ref[0]


def _tc4_body(u_ref, s_ref, xs3_ref, Wc_ref, bc_ref, crit_ref, h_ref):
    u = u_ref[0] + u_ref[1]
    s0 = s_ref[0, :, 0:1] + s_ref[1, :, 0:1]
    h = u / (s0 + 1e-16) + xs3_ref[...]
    h_ref[...] = h
    crit_ref[...] = jnp.dot(h, Wc_ref[...],
                            preferred_element_type=f32) + bc_ref[0]


def _row_spec(*shape):
    nd = len(shape)

    def im(i):
        return tuple(0 for _ in range(nd - 2)) + (i, 0)

    return pl.BlockSpec(shape, im)


def _full_spec(*shape):
    nd = len(shape)

    def im(i):
        return tuple(0 for _ in range(nd))

    return pl.BlockSpec(shape, im)


_zk16 = _make_zkernel(16)
_zk32 = _make_zkernel(32)
_sk_l0 = _make_scale_kernel(8, 1, 16)
_sk_l1 = _make_scale_kernel(4, 2, 16)
_sk_l2 = _make_scale_kernel(8, 2, 32)


def kernel(nodes, edges, edge_type, emb, Wm0, bm0, We0, be0, att0, Wm1, bm1,
           We1, be1, att1, Ws1, bs1, Wm2, bm2, We2, be2, att2, Wm3, bm3,
           att3, Ws3, bs3, Wc, bc):
    src_r = edges[0].astype(i32).reshape(NT, NBLK, NSUB, SUB)
    dst_r = edges[1].astype(i32).reshape(NT, NBLK, NSUB, SUB)
    src_s = edges[0].astype(i32).reshape(NT, EPT // 80, 80)
    dst_s = edges[1].astype(i32).reshape(NT, EPT // 80, 80)
    et = edge_type.astype(f32)

    b0 = (bm0 + be0).reshape(1, 1024)
    b1 = (bm1 + be1).reshape(1, 512)
    b2 = (bm2 + be2).reshape(1, 1024)
    awe0 = jnp.sum(We0.reshape(8, 128) * att0, axis=-1)
    awe0 = jnp.concatenate([awe0, awe0])
    awe1 = jnp.sum(We1.reshape(8, 64) * att1, axis=-1)
    awe1 = jnp.concatenate([awe1, awe1])
    awe2 = jnp.sum(We2.reshape(16, 64) * att2, axis=-1)
    nodes3 = jnp.pad(nodes.astype(i32), (0, NP - N)).reshape(GRID, 1, BN)

    y0, a0, x0 = pl.pallas_call(
        _tc0_body,
        grid=(GRID,),
        in_specs=[
            pl.BlockSpec((1, 1, BN), lambda i: (i, 0, 0)),
            _full_spec(50, 128),
            _full_spec(128, 1024),
            _full_spec(1, 1024),
            _full_spec(8, 128),
        ],
        out_specs=[
            _row_spec(8, BN, 128),
            _row_spec(BN, 16),
            _row_spec(BN, 128),
        ],
        out_shape=[
            jax.ShapeDtypeStruct((8, NP, 128), f32),
            jax.ShapeDtypeStruct((NP, 16), f32),
            jax.ShapeDtypeStruct((NP, 128), f32),
        ],
    )(nodes3, emb, Wm0, b0, att0)

    z0, s0p = _zk16(a0, et, src_r, dst_r, awe0)
    u0 = _sk_l0(y0, z0, src_s, dst_s)

    y1, a1, xs1 = pl.pallas_call(
        _tc1_body,
        grid=(GRID,),
        in_specs=[
            _row_spec(2, 8, BN, 128),
            _row_spec(2, BN, 16),
            _row_spec(BN, 128),
            _full_spec(1, 1024),
            _full_spec(128, 512),
            _full_spec(1, 512),
            _full_spec(8, 64),
            _full_spec(128, 64),
            _full_spec(1, 64),
        ],
        out_specs=[
            _row_spec(4, BN, 128),
            _row_spec(BN, 16),
            _row_spec(BN, 64),
        ],
        out_shape=[
            jax.ShapeDtypeStruct((4, NP, 128), f32),
            jax.ShapeDtypeStruct((NP, 16), f32),
            jax.ShapeDtypeStruct((NP, 64), f32),
        ],
    )(u0, s0p, x0, We0, Wm1, b1, att1, Ws1, bs1.reshape(1, 64))

    z1, s1p = _zk16(a1, et, src_r, dst_r, awe1)
    u1 = _sk_l1(y1, z1, src_s, dst_s)

    y2, a2, x2 = pl.pallas_call(
        _tc2_body,
        grid=(GRID,),
        in_specs=[
            _row_spec(2, 4, BN, 128),
            _row_spec(2, BN, 16),
            _row_spec(BN, 64),
            _full_spec(1, 512),
            _full_spec(64, 1024),
            _full_spec(1, 1024),
            _full_spec(16, 64),
        ],
        out_specs=[
            _row_spec(8, BN, 128),
            _row_spec(BN, 16),
            _row_spec(BN, 64),
        ],
        out_shape=[
            jax.ShapeDtypeStruct((8, NP, 128), f32),
            jax.ShapeDtypeStruct((NP, 16), f32),
            jax.ShapeDtypeStruct((NP, 64), f32),
        ],
    )(u1, s1p, xs1, We1, Wm2, b2, att2)

    z2, s2p = _zk32(a2, et, src_r, dst_r, awe2)
    u2 = _sk_l2(y2, z2, src_s, dst_s)

    y3, a3, xs3 = pl.pallas_call(
        _tc3_body,
        grid=(GRID,),
        in_specs=[
            _row_spec(2, 8, BN, 128),
            _row_spec(2, BN, 32),
            _row_spec(BN, 64),
            _full_spec(1, 1024),
            _full_spec(64, 32),
            _full_spec(1, 32),
            _full_spec(1, 32),
            _full_spec(64, 32),
            _full_spec(1, 32),
        ],
        out_specs=[
            _row_spec(BN, 32),
            _row_spec(BN, 16),
            _row_spec(BN, 32),
        ],
        out_shape=[
            jax.ShapeDtypeStruct((NP, 32), f32),
            jax.ShapeDtypeStruct((NP, 16), f32),
            jax.ShapeDtypeStruct((NP, 32), f32),
        ],
    )(u2, s2p, x2, We2, Wm3, bm3.reshape(1, 32), att3, Ws3,
      bs3.reshape(1, 32))

    u3, s3p = _l3_kernel(a3, y3, src_r, dst_r)

    critic, h = pl.pallas_call(
        _tc4_body,
        grid=(GRID,),
        in_specs=[
            _row_spec(2, BN, 32),
            _row_spec(2, BN, 16),
            _row_spec(BN, 32),
            _full_spec(32, 1),
            _full_spec(1, 1),
        ],
        out_specs=[
            _row_spec(BN, 1),
            _row_spec(BN, 32),
        ],
        out_shape=[
            jax.ShapeDtypeStruct((NP, 1), f32),
            jax.ShapeDtypeStruct((NP, 32), f32),
        ],
    )(u3, s3p, xs3, Wc, bc.reshape(1, 1))

    return (critic[:N], h[:N])
